# Initial kernel scaffold; baseline (speedup 1.0000x reference)
import jax
import jax.numpy as jnp
from jax.experimental import pallas as pl


def kernel(x, edge_index, W_l, W_r, b):
    raise NotImplementedError("write your pallas kernel here")


# speedup vs baseline: 6.0523x; 6.0523x over previous
import functools

import jax
import jax.numpy as jnp
from jax import lax
from jax.experimental import pallas as pl
from jax.experimental.pallas import tpu as pltpu
from jax.experimental.pallas import tpu_sc as plsc

_N = 10000
_E = 320000
_D = 128

_NC = 2
_NS = 16
_NW = _NC * _NS
_C = 80
_EPW = _E // _NW
_CHUNKS = _EPW // _C

_ROWS_PER_SUB = _N // _NS
_CNT_SUBS = 10


def _agg_body(x_hbm, src_hbm, dst_hbm, z2d_hbm,
              acc_out, cnt_out,
              src_i, dst_i, rows_v, ones_v, zcnt_v, acc_sh, cnt_sh, sem):
  c = lax.axis_index("c")
  s = lax.axis_index("s")
  wid = s * _NC + c

  for i in range(_C // 16):
    ones_v[pl.ds(i * 16, 16)] = jnp.ones((16,), jnp.float32)
  for i in range(1024 // 16):
    zcnt_v[pl.ds(i * 16, 16)] = jnp.zeros((16,), jnp.float32)

  @pl.when(s < _CNT_SUBS)
  def _():
    pltpu.sync_copy(z2d_hbm.at[pl.ds(s * 1000, 1000)],
                    acc_sh.at[pl.ds(s * 1000, 1000)])
    pltpu.sync_copy(zcnt_v.at[pl.ds(0, 1000)],
                    cnt_sh.at[pl.ds(s * 1000, 1000)])

  plsc.subcore_barrier()

  base = wid * _EPW

  def chunk_body(k, carry):
    off = base + k * _C
    pltpu.sync_copy(src_hbm.at[pl.ds(off, _C)], src_i)
    pltpu.sync_copy(dst_hbm.at[pl.ds(off, _C)], dst_i)
    pltpu.async_copy(x_hbm.at[src_i], rows_v, sem).wait()
    pltpu.sync_copy(rows_v, acc_sh.at[dst_i], add=True)
    pltpu.sync_copy(ones_v, cnt_sh.at[dst_i], add=True)
    return carry

  lax.fori_loop(0, _CHUNKS, chunk_body, 0)

  plsc.subcore_barrier()

  @pl.when(s < _CNT_SUBS)
  def _():
    pltpu.sync_copy(acc_sh.at[pl.ds(s * 1000, 1000)],
                    acc_out.at[c, pl.ds(s * 1000, 1000)])
    pltpu.sync_copy(cnt_sh.at[pl.ds(s * 1000, 1000)],
                    zcnt_v.at[pl.ds(0, 1000)])
    pltpu.sync_copy(zcnt_v.at[pl.ds(0, 1000)],
                    cnt_out.at[pl.ds(c * _N + s * 1000, 1000)])


_agg = pl.kernel(
    _agg_body,
    out_type=(
        jax.ShapeDtypeStruct((_NC, _N, _D), jnp.float32),
        jax.ShapeDtypeStruct((_NC * _N,), jnp.float32),
    ),
    mesh=plsc.VectorSubcoreMesh(
        core_axis_name="c", subcore_axis_name="s",
        num_cores=_NC, num_subcores=_NS),
    scratch_types=[
        pltpu.VMEM((_C,), jnp.int32),
        pltpu.VMEM((_C,), jnp.int32),
        pltpu.VMEM((_C, _D), jnp.float32),
        pltpu.VMEM((_C,), jnp.float32),
        pltpu.VMEM((1024,), jnp.float32),
        pltpu.VMEM_SHARED((_N, _D), jnp.float32),
        pltpu.VMEM_SHARED((_N,), jnp.float32),
        pltpu.SemaphoreType.DMA,
    ],
)


def _combine_body(acc_ref, cnt_ref, x_ref, wl_ref, wr_ref, b_ref, o_ref):
  summed = acc_ref[0] + acc_ref[1]
  cnt = cnt_ref[0] + cnt_ref[1]
  mean = summed / jnp.maximum(cnt, 1.0)
  out = (jnp.dot(mean, wl_ref[...], preferred_element_type=jnp.float32)
         + jnp.dot(x_ref[...], wr_ref[...], preferred_element_type=jnp.float32)
         + b_ref[...])
  o_ref[...] = jnp.maximum(out, 0.0)


_R = 1000


def _combine(acc, cnt, x, W_l, W_r, b2d):
  grid = _N // _R
  return pl.pallas_call(
      _combine_body,
      grid=(grid,),
      in_specs=[
          pl.BlockSpec((_NC, _R, _D), lambda i: (0, i, 0)),
          pl.BlockSpec((_NC, _R, 1), lambda i: (0, i, 0)),
          pl.BlockSpec((_R, _D), lambda i: (i, 0)),
          pl.BlockSpec((_D, _D), lambda i: (0, 0)),
          pl.BlockSpec((_D, _D), lambda i: (0, 0)),
          pl.BlockSpec((1, _D), lambda i: (0, 0)),
      ],
      out_specs=pl.BlockSpec((_R, _D), lambda i: (i, 0)),
      out_shape=jax.ShapeDtypeStruct((_N, _D), jnp.float32),
  )(acc, cnt, x, W_l, W_r, b2d)


def kernel(x, edge_index, W_l, W_r, b):
  src = edge_index[0]
  dst = edge_index[1]
  z2d = jnp.zeros((_N, _D), jnp.float32)
  acc, cnt = _agg(x, src, dst, z2d)
  cnt = cnt.reshape(_NC, _N, 1)
  return _combine(acc, cnt, x, W_l, W_r, b.reshape(1, _D))
